# Initial kernel scaffold; baseline (speedup 1.0000x reference)
import jax
import jax.numpy as jnp
from jax.experimental import pallas as pl


def kernel(x, edge_index, edge_weight, W1, W2, W3):
    raise NotImplementedError("write your pallas kernel here")


# speedup vs baseline: 1.8331x; 1.8331x over previous
import dataclasses
import functools

import jax
import jax.numpy as jnp
from jax import lax
from jax.experimental import pallas as pl
from jax.experimental.pallas import tpu as pltpu
from jax.experimental.pallas import tpu_sc as plsc

N = 10000
E = 160000
LANES = 16
N_CORES = 2
N_SUBCORES = 16
EDGES_PER_TILE = E // N_SUBCORES
WIN = 80
N_PAD = 10240
ROWS_PER_TILE = N_PAD // N_SUBCORES
ZROWS = 32
MM_BM = 1000
DEC_BM = 1024


def _spmm_body(Dc, row_hbm, col_hbm, w_hbm, ha_hbm, hb_hbm, oa_hbm, ob_hbm,
               row_v, col_v, w_v, gath_v, zero_v, acc_sh, sem):
    c = lax.axis_index("c")
    t = lax.axis_index("s")
    zvec = jnp.zeros((LANES,), jnp.float32)

    @pl.loop(0, ZROWS)
    def _(r):
        @pl.loop(0, Dc, step=LANES)
        def _(d):
            zero_v[r, pl.ds(d, LANES)] = zvec

    @pl.loop(0, ROWS_PER_TILE, step=ZROWS)
    def _(r):
        pltpu.sync_copy(zero_v, acc_sh.at[pl.ds(t * ROWS_PER_TILE + r, ZROWS)])

    plsc.subcore_barrier()

    for ci in range(N_CORES):
        h_ref = (ha_hbm, hb_hbm)[ci]

        @pl.when(c == ci)
        def _(h_ref=h_ref):
            @pl.loop(0, EDGES_PER_TILE, step=WIN)
            def _(j):
                base = t * EDGES_PER_TILE + j
                pltpu.sync_copy(row_hbm.at[pl.ds(base, WIN)], row_v)
                pltpu.sync_copy(col_hbm.at[pl.ds(base, WIN)], col_v)
                pltpu.sync_copy(w_hbm.at[pl.ds(base, WIN)], w_v)
                pltpu.async_copy(h_ref.at[col_v], gath_v, sem).wait()

                @pl.loop(0, WIN)
                def _(e):
                    idx16 = jnp.full((LANES,), e, jnp.int32)
                    wvec = plsc.load_gather(w_v, [idx16])

                    @pl.loop(0, Dc, step=LANES)
                    def _(d):
                        gath_v[e, pl.ds(d, LANES)] = (
                            gath_v[e, pl.ds(d, LANES)] * wvec)

                pltpu.sync_copy(gath_v, acc_sh.at[row_v], add=True)

    plsc.subcore_barrier()

    for ci in range(N_CORES):
        o_ref = (oa_hbm, ob_hbm)[ci]

        @pl.when(c == ci)
        def _(o_ref=o_ref):
            @pl.when(t < N_SUBCORES - 1)
            def _():
                sl = pl.ds(t * ROWS_PER_TILE, ROWS_PER_TILE)
                pltpu.sync_copy(acc_sh.at[sl], o_ref.at[sl])

            @pl.when(t == N_SUBCORES - 1)
            def _():
                tail = N - (N_SUBCORES - 1) * ROWS_PER_TILE
                sl = pl.ds((N_SUBCORES - 1) * ROWS_PER_TILE, tail)
                pltpu.sync_copy(acc_sh.at[sl], o_ref.at[sl])


@functools.cache
def _make_spmm(Dc):
    mesh = plsc.VectorSubcoreMesh(core_axis_name="c", subcore_axis_name="s")
    cp = pltpu.CompilerParams()
    if "needs_layout_passes" in pltpu.CompilerParams.__dataclass_fields__:
        cp = dataclasses.replace(cp, needs_layout_passes=False)
    if "use_tc_tiling_on_sc" in pltpu.CompilerParams.__dataclass_fields__:
        cp = dataclasses.replace(cp, use_tc_tiling_on_sc=False)
    return pl.kernel(
        functools.partial(_spmm_body, Dc),
        compiler_params=cp,
        out_type=[jax.ShapeDtypeStruct((N, Dc), jnp.float32)] * 2,
        mesh=mesh,
        scratch_types=[
            pltpu.VMEM((WIN,), jnp.int32),
            pltpu.VMEM((WIN,), jnp.int32),
            pltpu.VMEM((WIN,), jnp.float32),
            pltpu.VMEM((WIN, Dc), jnp.float32),
            pltpu.VMEM((ZROWS, Dc), jnp.float32),
            pltpu.VMEM_SHARED((N_PAD, Dc), jnp.float32),
            pltpu.SemaphoreType.DMA,
        ],
    )


def _spmm(row, col, w, ha, hb):
    return _make_spmm(ha.shape[1])(row, col, w, ha, hb)


def _dot_default(a, b, dims):
    return lax.dot_general(a.astype(jnp.bfloat16), b.astype(jnp.bfloat16),
                           dims, preferred_element_type=jnp.float32)


_MM_DIMS = (((1,), (0,)), ((), ()))
_DEC_DIMS = (((1,), (1,)), ((), ()))


def _mm_body(nx, act, refs):
    acc = None
    for i in range(nx):
        p = _dot_default(refs[i][...], refs[nx + i][...], _MM_DIMS)
        acc = p if acc is None else acc + p
    if act:
        acc = jnp.tanh(acc)
    h = acc.shape[-1] // 2
    refs[2 * nx][...] = acc[:, :h]
    refs[2 * nx + 1][...] = acc[:, h:]


def _mm_chunks(xs, ws, act):
    nx = len(xs)
    dout = ws[0].shape[1]
    in_specs = (
        [pl.BlockSpec((MM_BM, x.shape[1]), lambda i: (i, 0)) for x in xs]
        + [pl.BlockSpec(w.shape, lambda i: (0, 0)) for w in ws])
    return pl.pallas_call(
        lambda *rs: _mm_body(nx, act, rs),
        grid=(N // MM_BM,),
        in_specs=in_specs,
        out_specs=[pl.BlockSpec((MM_BM, dout // 2), lambda i: (i, 0))] * 2,
        out_shape=[jax.ShapeDtypeStruct((N, dout // 2), jnp.float32)] * 2,
    )(*xs, *ws)


def _dec_body(zi_ref, zj_ref, o_ref):
    logits = _dot_default(zi_ref[...], zj_ref[...], _DEC_DIMS)
    o_ref[...] = jax.nn.sigmoid(logits)


def _decoder(z):
    d = z.shape[1]
    nb = pl.cdiv(N, DEC_BM)
    return pl.pallas_call(
        _dec_body,
        grid=(nb, nb),
        in_specs=[pl.BlockSpec((DEC_BM, d), lambda i, j: (i, 0)),
                  pl.BlockSpec((DEC_BM, d), lambda i, j: (j, 0))],
        out_specs=pl.BlockSpec((DEC_BM, DEC_BM), lambda i, j: (i, j)),
        out_shape=jax.ShapeDtypeStruct((N, N), jnp.float32),
    )(z, z)


def kernel(x, edge_index, edge_weight, W1, W2, W3):
    row = edge_index[0]
    col = edge_index[1]

    s1a, s1b = _mm_chunks([x], [W1], act=True)
    z1a, z1b = _spmm(row, col, edge_weight, s1a, s1b)
    az1a, az1b = _spmm(row, col, edge_weight, z1a, z1b)

    s2a, s2b = _mm_chunks([z1a, z1b], [W2[:128], W2[128:]], act=True)
    z2a, z2b = _spmm(row, col, edge_weight, s2a, s2b)
    az2a, az2b = _spmm(row, col, edge_weight, z2a, z2b)

    s3a, s3b = _mm_chunks([z2a, z2b], [W3[:64], W3[64:]], act=False)
    za, zb = _spmm(row, col, edge_weight, s3a, s3b)
    az3a, az3b = _spmm(row, col, edge_weight, za, zb)

    z_igae = jnp.concatenate([za, zb], axis=1)
    z_igae_adj = _decoder(z_igae)

    z1 = jnp.concatenate([z1a, z1b], axis=1)
    z2 = jnp.concatenate([z2a, z2b], axis=1)
    az1 = jnp.concatenate([az1a, az1b], axis=1)
    az2 = jnp.concatenate([az2a, az2b], axis=1)
    az3 = jnp.concatenate([az3a, az3b], axis=1)
    return (z_igae, z_igae_adj, az1, az2, az3, z1, z2)


# speedup vs baseline: 4.7622x; 2.5979x over previous
import dataclasses
import functools

import jax
import jax.numpy as jnp
from jax import lax
from jax.experimental import pallas as pl
from jax.experimental.pallas import tpu as pltpu
from jax.experimental.pallas import tpu_sc as plsc

N = 10000
E = 160000
LANES = 16
N_CORES = 2
N_SUBCORES = 16
EDGES_PER_TILE = E // N_SUBCORES
WIN = 80
N_PAD = 10240
ROWS_PER_TILE = N_PAD // N_SUBCORES
ZROWS = 8
MM_BM = 1000
DEC_BM = 1024


def _spmm_body(Dc, row_hbm, col_hbm, w_hbm, ha_hbm, hb_hbm, oa_hbm, ob_hbm,
               col_t, gath_v0, gath_v1, row_v0, row_v1, w_v0, w_v1,
               zero_v, acc_sh, sem0, sem1, isem):
    c = lax.axis_index("c")
    t = lax.axis_index("s")
    zvec = jnp.zeros((LANES,), jnp.float32)
    n_win = EDGES_PER_TILE // WIN

    cc = pltpu.async_copy(col_hbm.at[t], col_t, isem)

    @pl.loop(0, ZROWS)
    def _(r):
        @pl.loop(0, Dc, step=LANES)
        def _(d):
            zero_v[r, pl.ds(d, LANES)] = zvec

    @pl.loop(0, ROWS_PER_TILE, step=ZROWS)
    def _(r):
        pltpu.sync_copy(zero_v, acc_sh.at[pl.ds(t * ROWS_PER_TILE + r, ZROWS)])

    cc.wait()
    plsc.subcore_barrier()

    for ci in range(N_CORES):
        h_ref = (ha_hbm, hb_hbm)[ci]

        @pl.when(c == ci)
        def _(h_ref=h_ref):
            def start(j, gath_v, row_v, w_v, sem):
                base = t * EDGES_PER_TILE + j * WIN
                pltpu.async_copy(h_ref.at[col_t.at[j]], gath_v, sem)
                pltpu.async_copy(row_hbm.at[pl.ds(base, WIN)], row_v, sem)
                pltpu.async_copy(w_hbm.at[pl.ds(base, WIN)], w_v, sem)

            def process(j, gath_v, row_v, w_v, sem):
                base = t * EDGES_PER_TILE + j * WIN
                pltpu.make_async_copy(h_ref.at[col_t.at[j]], gath_v,
                                      sem).wait()
                pltpu.make_async_copy(row_hbm.at[pl.ds(base, WIN)], row_v,
                                      sem).wait()
                pltpu.make_async_copy(w_hbm.at[pl.ds(base, WIN)], w_v,
                                      sem).wait()

                @pl.loop(0, WIN, step=LANES)
                def _(e):
                    wvec = w_v[pl.ds(e, LANES)]
                    for k in range(LANES):
                        wk = wvec[k]
                        for d in range(0, Dc, LANES):
                            sl = pl.ds(d, LANES)
                            gath_v[e + k, sl] = gath_v[e + k, sl] * wk

                pltpu.sync_copy(gath_v, acc_sh.at[row_v], add=True)

            start(0, gath_v0, row_v0, w_v0, sem0)

            @pl.loop(0, n_win - 1, step=2)
            def _(j):
                start(j + 1, gath_v1, row_v1, w_v1, sem1)
                process(j, gath_v0, row_v0, w_v0, sem0)
                start(j + 2, gath_v0, row_v0, w_v0, sem0)
                process(j + 1, gath_v1, row_v1, w_v1, sem1)

            process(n_win - 1, gath_v0, row_v0, w_v0, sem0)

    plsc.subcore_barrier()

    for ci in range(N_CORES):
        o_ref = (oa_hbm, ob_hbm)[ci]

        @pl.when(c == ci)
        def _(o_ref=o_ref):
            @pl.when(t < N_SUBCORES - 1)
            def _():
                sl = pl.ds(t * ROWS_PER_TILE, ROWS_PER_TILE)
                pltpu.sync_copy(acc_sh.at[sl], o_ref.at[sl])

            @pl.when(t == N_SUBCORES - 1)
            def _():
                tail = N - (N_SUBCORES - 1) * ROWS_PER_TILE
                sl = pl.ds((N_SUBCORES - 1) * ROWS_PER_TILE, tail)
                pltpu.sync_copy(acc_sh.at[sl], o_ref.at[sl])


@functools.cache
def _make_spmm(Dc):
    mesh = plsc.VectorSubcoreMesh(core_axis_name="c", subcore_axis_name="s")
    cp = pltpu.CompilerParams()
    if "needs_layout_passes" in pltpu.CompilerParams.__dataclass_fields__:
        cp = dataclasses.replace(cp, needs_layout_passes=False)
    if "use_tc_tiling_on_sc" in pltpu.CompilerParams.__dataclass_fields__:
        cp = dataclasses.replace(cp, use_tc_tiling_on_sc=False)
    return pl.kernel(
        functools.partial(_spmm_body, Dc),
        compiler_params=cp,
        out_type=[jax.ShapeDtypeStruct((N, Dc), jnp.float32)] * 2,
        mesh=mesh,
        scratch_types=[
            pltpu.VMEM((EDGES_PER_TILE // WIN, WIN), jnp.int32),
            pltpu.VMEM((WIN, Dc), jnp.float32),
            pltpu.VMEM((WIN, Dc), jnp.float32),
            pltpu.VMEM((WIN,), jnp.int32),
            pltpu.VMEM((WIN,), jnp.int32),
            pltpu.VMEM((WIN,), jnp.float32),
            pltpu.VMEM((WIN,), jnp.float32),
            pltpu.VMEM((ZROWS, Dc), jnp.float32),
            pltpu.VMEM_SHARED((N_PAD, Dc), jnp.float32),
            pltpu.SemaphoreType.DMA,
            pltpu.SemaphoreType.DMA,
            pltpu.SemaphoreType.DMA,
        ],
    )


def _spmm(row, col, w, ha, hb):
    shape = (N_SUBCORES, EDGES_PER_TILE // WIN, WIN)
    return _make_spmm(ha.shape[1])(row, col.reshape(shape), w, ha, hb)


def _dot_default(a, b, dims):
    return lax.dot_general(a.astype(jnp.bfloat16), b.astype(jnp.bfloat16),
                           dims, preferred_element_type=jnp.float32)


_MM_DIMS = (((1,), (0,)), ((), ()))
_DEC_DIMS = (((1,), (1,)), ((), ()))


def _mm_body(nx, act, refs):
    acc = None
    for i in range(nx):
        p = _dot_default(refs[i][...], refs[nx + i][...], _MM_DIMS)
        acc = p if acc is None else acc + p
    if act:
        acc = jnp.tanh(acc)
    h = acc.shape[-1] // 2
    refs[2 * nx][...] = acc[:, :h]
    refs[2 * nx + 1][...] = acc[:, h:]


def _mm_chunks(xs, ws, act):
    nx = len(xs)
    dout = ws[0].shape[1]
    in_specs = (
        [pl.BlockSpec((MM_BM, x.shape[1]), lambda i: (i, 0)) for x in xs]
        + [pl.BlockSpec(w.shape, lambda i: (0, 0)) for w in ws])
    return pl.pallas_call(
        lambda *rs: _mm_body(nx, act, rs),
        grid=(N // MM_BM,),
        in_specs=in_specs,
        out_specs=[pl.BlockSpec((MM_BM, dout // 2), lambda i: (i, 0))] * 2,
        out_shape=[jax.ShapeDtypeStruct((N, dout // 2), jnp.float32)] * 2,
    )(*xs, *ws)


def _dec_body(zi_ref, zj_ref, o_ref):
    logits = _dot_default(zi_ref[...], zj_ref[...], _DEC_DIMS)
    o_ref[...] = jax.nn.sigmoid(logits)


def _decoder(z):
    d = z.shape[1]
    nb = pl.cdiv(N, DEC_BM)
    return pl.pallas_call(
        _dec_body,
        grid=(nb, nb),
        in_specs=[pl.BlockSpec((DEC_BM, d), lambda i, j: (i, 0)),
                  pl.BlockSpec((DEC_BM, d), lambda i, j: (j, 0))],
        out_specs=pl.BlockSpec((DEC_BM, DEC_BM), lambda i, j: (i, j)),
        out_shape=jax.ShapeDtypeStruct((N, N), jnp.float32),
    )(z, z)


def kernel(x, edge_index, edge_weight, W1, W2, W3):
    row = edge_index[0]
    col = edge_index[1]

    s1a, s1b = _mm_chunks([x], [W1], act=True)
    z1a, z1b = _spmm(row, col, edge_weight, s1a, s1b)
    az1a, az1b = _spmm(row, col, edge_weight, z1a, z1b)

    s2a, s2b = _mm_chunks([z1a, z1b], [W2[:128], W2[128:]], act=True)
    z2a, z2b = _spmm(row, col, edge_weight, s2a, s2b)
    az2a, az2b = _spmm(row, col, edge_weight, z2a, z2b)

    s3a, s3b = _mm_chunks([z2a, z2b], [W3[:64], W3[64:]], act=False)
    za, zb = _spmm(row, col, edge_weight, s3a, s3b)
    az3a, az3b = _spmm(row, col, edge_weight, za, zb)

    z_igae = jnp.concatenate([za, zb], axis=1)
    z_igae_adj = _decoder(z_igae)

    z1 = jnp.concatenate([z1a, z1b], axis=1)
    z2 = jnp.concatenate([z2a, z2b], axis=1)
    az1 = jnp.concatenate([az1a, az1b], axis=1)
    az2 = jnp.concatenate([az2a, az2b], axis=1)
    az3 = jnp.concatenate([az3a, az3b], axis=1)
    return (z_igae, z_igae_adj, az1, az2, az3, z1, z2)


# speedup vs baseline: 5.6230x; 1.1807x over previous
import dataclasses
import functools

import jax
import jax.numpy as jnp
from jax import lax
from jax.experimental import pallas as pl
from jax.experimental.pallas import tpu as pltpu
from jax.experimental.pallas import tpu_sc as plsc

N = 10000
E = 160000
LANES = 16
N_CORES = 2
N_SUBCORES = 16
EDGES_PER_TILE = E // N_SUBCORES
WIN = 80
N_PAD = 10240
ROWS_PER_TILE = N_PAD // N_SUBCORES
ZROWS = 8
MM_BM = 1000
DEC_BM = 1024


def _scale_window(gath_v, w_v, Dc, win):
    def block(base, k_lo):
        wvec = w_v[pl.ds(base, LANES)]
        for k in range(k_lo, LANES):
            wk = wvec[k]
            for d in range(0, Dc, LANES):
                sl = pl.ds(d, LANES)
                gath_v[base + k, sl] = gath_v[base + k, sl] * wk

    e0 = 0
    while e0 + LANES <= win:
        block(e0, 0)
        e0 += LANES
    if e0 < win:
        block(win - LANES, LANES - (win - e0))


def _spmm_body(Dc, edge_split, row_hbm, col_hbm, w_hbm, ha_hbm, hb_hbm,
               oa_hbm, ob_hbm,
               col_t, gath_v0, gath_v1, row_v0, row_v1, w_v0, w_v1,
               zero_v, acc_sh, sem0, sem1, isem):
    c = lax.axis_index("c")
    t = lax.axis_index("s")
    zvec = jnp.zeros((LANES,), jnp.float32)
    if edge_split:
        ept = E // (N_CORES * N_SUBCORES)
        win = 40
        wid = c * N_SUBCORES + t
    else:
        ept = EDGES_PER_TILE
        win = WIN
        wid = t
    n_win = ept // win

    cc = pltpu.async_copy(col_hbm.at[wid], col_t, isem)

    @pl.loop(0, ZROWS)
    def _(r):
        @pl.loop(0, Dc, step=LANES)
        def _(d):
            zero_v[r, pl.ds(d, LANES)] = zvec

    @pl.loop(0, ROWS_PER_TILE, step=ZROWS)
    def _(r):
        pltpu.sync_copy(zero_v, acc_sh.at[pl.ds(t * ROWS_PER_TILE + r, ZROWS)])

    cc.wait()
    plsc.subcore_barrier()

    def edge_loop(h_ref):
        def start(j, gath_v, row_v, w_v, sem):
            base = wid * ept + j * win
            pltpu.async_copy(h_ref.at[col_t.at[j]], gath_v, sem)
            pltpu.async_copy(row_hbm.at[pl.ds(base, win)], row_v, sem)
            pltpu.async_copy(w_hbm.at[pl.ds(base, win)], w_v, sem)

        def process(j, gath_v, row_v, w_v, sem):
            base = wid * ept + j * win
            pltpu.make_async_copy(h_ref.at[col_t.at[j]], gath_v,
                                  sem).wait()
            pltpu.make_async_copy(row_hbm.at[pl.ds(base, win)], row_v,
                                  sem).wait()
            pltpu.make_async_copy(w_hbm.at[pl.ds(base, win)], w_v,
                                  sem).wait()
            _scale_window(gath_v, w_v, Dc, win)
            pltpu.sync_copy(gath_v, acc_sh.at[row_v], add=True)

        start(0, gath_v0, row_v0, w_v0, sem0)

        @pl.loop(0, n_win - 1, step=2)
        def _(j):
            start(j + 1, gath_v1, row_v1, w_v1, sem1)
            process(j, gath_v0, row_v0, w_v0, sem0)
            start(j + 2, gath_v0, row_v0, w_v0, sem0)
            process(j + 1, gath_v1, row_v1, w_v1, sem1)

        process(n_win - 1, gath_v0, row_v0, w_v0, sem0)

    if edge_split:
        edge_loop(ha_hbm)
    else:
        for ci in range(N_CORES):
            h_ref = (ha_hbm, hb_hbm)[ci]

            @pl.when(c == ci)
            def _(h_ref=h_ref):
                edge_loop(h_ref)

    plsc.subcore_barrier()

    for ci in range(N_CORES):
        o_ref = (oa_hbm, ob_hbm)[ci]

        @pl.when(c == ci)
        def _(o_ref=o_ref):
            @pl.when(t < N_SUBCORES - 1)
            def _():
                sl = pl.ds(t * ROWS_PER_TILE, ROWS_PER_TILE)
                pltpu.sync_copy(acc_sh.at[sl], o_ref.at[sl])

            @pl.when(t == N_SUBCORES - 1)
            def _():
                tail = N - (N_SUBCORES - 1) * ROWS_PER_TILE
                sl = pl.ds((N_SUBCORES - 1) * ROWS_PER_TILE, tail)
                pltpu.sync_copy(acc_sh.at[sl], o_ref.at[sl])


@functools.cache
def _make_spmm(Dc, edge_split=False):
    mesh = plsc.VectorSubcoreMesh(core_axis_name="c", subcore_axis_name="s")
    cp = pltpu.CompilerParams()
    if "needs_layout_passes" in pltpu.CompilerParams.__dataclass_fields__:
        cp = dataclasses.replace(cp, needs_layout_passes=False)
    if "use_tc_tiling_on_sc" in pltpu.CompilerParams.__dataclass_fields__:
        cp = dataclasses.replace(cp, use_tc_tiling_on_sc=False)
    win = 40 if edge_split else WIN
    n_win = (E // (N_CORES * N_SUBCORES) if edge_split
             else EDGES_PER_TILE) // win
    return pl.kernel(
        functools.partial(_spmm_body, Dc, edge_split),
        compiler_params=cp,
        out_type=[jax.ShapeDtypeStruct((N, Dc), jnp.float32)] * 2,
        mesh=mesh,
        scratch_types=[
            pltpu.VMEM((n_win, win), jnp.int32),
            pltpu.VMEM((win, Dc), jnp.float32),
            pltpu.VMEM((win, Dc), jnp.float32),
            pltpu.VMEM((win,), jnp.int32),
            pltpu.VMEM((win,), jnp.int32),
            pltpu.VMEM((win,), jnp.float32),
            pltpu.VMEM((win,), jnp.float32),
            pltpu.VMEM((ZROWS, Dc), jnp.float32),
            pltpu.VMEM_SHARED((N_PAD, Dc), jnp.float32),
            pltpu.SemaphoreType.DMA,
            pltpu.SemaphoreType.DMA,
            pltpu.SemaphoreType.DMA,
        ],
    )


def _spmm(row, col, w, ha, hb):
    shape = (N_SUBCORES, EDGES_PER_TILE // WIN, WIN)
    return _make_spmm(ha.shape[1])(row, col.reshape(shape), w, ha, hb)


def _spmm_es(row, col, w, h):
    nw = N_CORES * N_SUBCORES
    win = 40
    shape = (nw, (E // nw) // win, win)
    return _make_spmm(h.shape[1], True)(row, col.reshape(shape), w, h, h)


def _add_body(a_ref, b_ref, o_ref):
    o_ref[...] = a_ref[...] + b_ref[...]


def _padd(a, b):
    d = a.shape[1]
    return pl.pallas_call(
        _add_body,
        grid=(N // MM_BM,),
        in_specs=[pl.BlockSpec((MM_BM, d), lambda i: (i, 0))] * 2,
        out_specs=pl.BlockSpec((MM_BM, d), lambda i: (i, 0)),
        out_shape=jax.ShapeDtypeStruct((N, d), jnp.float32),
    )(a, b)


def _dot_default(a, b, dims):
    return lax.dot_general(a.astype(jnp.bfloat16), b.astype(jnp.bfloat16),
                           dims, preferred_element_type=jnp.float32)


_MM_DIMS = (((1,), (0,)), ((), ()))
_DEC_DIMS = (((1,), (1,)), ((), ()))


def _mm_body(nx, act, nout, refs):
    acc = None
    for i in range(nx):
        p = _dot_default(refs[i][...], refs[nx + i][...], _MM_DIMS)
        acc = p if acc is None else acc + p
    if act:
        acc = jnp.tanh(acc)
    if nout == 1:
        refs[2 * nx][...] = acc
    else:
        h = acc.shape[-1] // 2
        refs[2 * nx][...] = acc[:, :h]
        refs[2 * nx + 1][...] = acc[:, h:]


def _mm_chunks(xs, ws, act, nout=2):
    nx = len(xs)
    dout = ws[0].shape[1]
    in_specs = (
        [pl.BlockSpec((MM_BM, x.shape[1]), lambda i: (i, 0)) for x in xs]
        + [pl.BlockSpec(w.shape, lambda i: (0, 0)) for w in ws])
    return pl.pallas_call(
        lambda *rs: _mm_body(nx, act, nout, rs),
        grid=(N // MM_BM,),
        in_specs=in_specs,
        out_specs=[pl.BlockSpec((MM_BM, dout // nout),
                                lambda i: (i, 0))] * nout,
        out_shape=[jax.ShapeDtypeStruct((N, dout // nout),
                                        jnp.float32)] * nout,
    )(*xs, *ws)


def _dec_body(zi_ref, zj_ref, o_ref):
    logits = _dot_default(zi_ref[...], zj_ref[...], _DEC_DIMS)
    o_ref[...] = jax.nn.sigmoid(logits)


def _decoder(z):
    d = z.shape[1]
    nb = pl.cdiv(N, DEC_BM)
    return pl.pallas_call(
        _dec_body,
        grid=(nb, nb),
        in_specs=[pl.BlockSpec((DEC_BM, d), lambda i, j: (i, 0)),
                  pl.BlockSpec((DEC_BM, d), lambda i, j: (j, 0))],
        out_specs=pl.BlockSpec((DEC_BM, DEC_BM), lambda i, j: (i, j)),
        out_shape=jax.ShapeDtypeStruct((N, N), jnp.float32),
    )(z, z)


def kernel(x, edge_index, edge_weight, W1, W2, W3):
    row = edge_index[0]
    col = edge_index[1]

    s1a, s1b = _mm_chunks([x], [W1], act=True)
    z1a, z1b = _spmm(row, col, edge_weight, s1a, s1b)
    az1a, az1b = _spmm(row, col, edge_weight, z1a, z1b)

    (s2,) = _mm_chunks([z1a, z1b], [W2[:128], W2[128:]], act=True, nout=1)
    z2 = _padd(*_spmm_es(row, col, edge_weight, s2))
    az2 = _padd(*_spmm_es(row, col, edge_weight, z2))

    s3a, s3b = _mm_chunks([z2], [W3], act=False)
    za, zb = _spmm(row, col, edge_weight, s3a, s3b)
    az3a, az3b = _spmm(row, col, edge_weight, za, zb)

    z_igae = jnp.concatenate([za, zb], axis=1)
    z_igae_adj = _decoder(z_igae)

    z1 = jnp.concatenate([z1a, z1b], axis=1)
    az1 = jnp.concatenate([az1a, az1b], axis=1)
    az3 = jnp.concatenate([az3a, az3b], axis=1)
    return (z_igae, z_igae_adj, az1, az2, az3, z1, z2)
